# baseline (speedup 1.0000x reference)
import functools

import jax
import jax.numpy as jnp
from jax import lax
from jax.experimental import pallas as pl
from jax.experimental.pallas import tpu as pltpu
from jax.experimental.pallas import tpu_sc as plsc

_N_CLS = 1000
_N_CTX = 16
_CTX_DIM = 768
_SEQ = 77
_P = 3
_HALF = _N_CTX // 2
_SUF = _SEQ - _P - _N_CTX
_NQ = _CTX_DIM // 16

_INFO = plsc.get_sparse_core_info()
_NC = _INFO.num_cores
_NS = _INFO.num_subcores
_NW = _NC * _NS
_STEPS = -(-_N_CLS // _NW)


def _copy_row(out_v, buf, dst_row, src_ref, src_row):
    for q in range(_NQ):
        out_v[buf, dst_row, pl.ds(16 * q, 16)] = src_ref[src_row,
                                                         pl.ds(16 * q, 16)]


def _read_list(pre_h, suf_h, out_v, buf, c):
    return (
        (pre_h.at[c], out_v.at[buf, pl.ds(0, _P)]),
        (suf_h.at[c, pl.ds(0, 48)], out_v.at[buf, pl.ds(16, 48)]),
        (suf_h.at[c, pl.ds(48, 8)], out_v.at[buf, pl.ds(64, 8)]),
        (suf_h.at[c, pl.ds(56, 2)], out_v.at[buf, pl.ds(72, 2)]),
    )


def _fire_reads(pre_h, suf_h, out_v, rsem, buf, c):
    for src, dst in _read_list(pre_h, suf_h, out_v, buf, c):
        pltpu.async_copy(src, dst, rsem)


def _wait_reads(pre_h, suf_h, out_v, rsem, buf, c):
    for src, dst in _read_list(pre_h, suf_h, out_v, buf, c):
        pltpu.make_async_copy(src, dst, rsem).wait()


def _assemble(out_v, ctx2_v, buf, nl):
    b = out_v.at[buf]
    for k in range(_SUF - 1, 0, -1):
        _copy_row(out_v, buf, 19 + k, b, 16 + k)
    _copy_row(out_v, buf, 11, b, 16)
    for r in range(1, 9):
        _copy_row(out_v, buf, 11 + r, b, 19 + r)

    def copy_ctx2(r, carry):
        _copy_row(out_v, buf, 11 + nl + r, ctx2_v, r)
        return carry

    lax.fori_loop(0, _HALF, copy_ctx2, 0)


def _sc_body(ctx_h, pre_h, suf_h, nl_h, out_h, out_v, ctx2_v, nl_v,
             rsem0, rsem1, wsem0, wsem1):
    wid = lax.axis_index("s") * _NC + lax.axis_index("c")

    pltpu.sync_copy(ctx_h, out_v.at[0, pl.ds(0, _N_CTX)])

    for r in range(_HALF):
        for q in range(_NQ):
            ctx2_v[r, pl.ds(16 * q, 16)] = out_v[0, _HALF + r,
                                                 pl.ds(16 * q, 16)]
        _copy_row(out_v, 1, _P + r, out_v.at[0], r)
    for r in range(_HALF - 1, -1, -1):
        _copy_row(out_v, 0, _P + r, out_v.at[0], r)
    pltpu.sync_copy(nl_h.at[wid], nl_v)

    nlv0 = nl_v[0, pl.ds(0, 16)]
    nlv1 = nl_v[0, pl.ds(16, 16)]
    iota = lax.iota(jnp.int32, 16)

    def nl_of(j):
        return (jnp.sum(jnp.where(iota == j, nlv0, 0)) +
                jnp.sum(jnp.where(iota == j - 16, nlv1, 0)))

    _fire_reads(pre_h, suf_h, out_v, rsem0, 0, wid)
    _fire_reads(pre_h, suf_h, out_v, rsem1, 1, _NW + wid)

    def step(j, carry):
        c = j * _NW + wid
        buf = j & 1
        even = buf == 0

        @pl.when(c < _N_CLS)
        def _():
            @pl.when(even)
            def _():
                _wait_reads(pre_h, suf_h, out_v, rsem0, 0, c)

            @pl.when(~even)
            def _():
                _wait_reads(pre_h, suf_h, out_v, rsem1, 1, c)

            _assemble(out_v, ctx2_v, buf, nl_of(j))

            @pl.when(even)
            def _():
                pltpu.async_copy(out_v.at[0], out_h.at[c], wsem0)

                @pl.when(j > 0)
                def _():
                    pltpu.make_async_copy(out_v.at[1], out_h.at[c],
                                          wsem1).wait()

                @pl.when((j > 0) & (c + _NW < _N_CLS))
                def _():
                    _fire_reads(pre_h, suf_h, out_v, rsem1, 1, c + _NW)

            @pl.when(~even)
            def _():
                pltpu.async_copy(out_v.at[1], out_h.at[c], wsem1)
                pltpu.make_async_copy(out_v.at[0], out_h.at[c],
                                      wsem0).wait()

                @pl.when(c + _NW < _N_CLS)
                def _():
                    _fire_reads(pre_h, suf_h, out_v, rsem0, 0, c + _NW)

        return carry

    lax.fori_loop(0, _STEPS, step, 0)

    @pl.when(wid < 8)
    def _():
        pltpu.make_async_copy(out_v.at[1], out_h.at[0], wsem1).wait()

    @pl.when(wid >= 8)
    def _():
        pltpu.make_async_copy(out_v.at[0], out_h.at[0], wsem0).wait()


_build = pl.kernel(
    _sc_body,
    out_type=jax.ShapeDtypeStruct((_N_CLS, _SEQ, _CTX_DIM), jnp.float32),
    mesh=plsc.VectorSubcoreMesh(core_axis_name="c", subcore_axis_name="s"),
    scratch_types=[
        pltpu.VMEM((2, _SEQ, _CTX_DIM), jnp.float32),
        pltpu.VMEM((_HALF, _CTX_DIM), jnp.float32),
        pltpu.VMEM((1, _STEPS), jnp.int32),
        pltpu.SemaphoreType.DMA,
        pltpu.SemaphoreType.DMA,
        pltpu.SemaphoreType.DMA,
        pltpu.SemaphoreType.DMA,
    ],
    compiler_params=pltpu.CompilerParams(needs_layout_passes=False),
)


@functools.partial(jax.jit)
def kernel(ctx, token_prefix, token_suffix, name_lens, tokenized_prompts):
    nl_pad = jnp.zeros((_STEPS * _NW,), jnp.int32).at[:_N_CLS].set(name_lens)
    nl_t = nl_pad.reshape(_STEPS, _NW).T.reshape(_NW, 1, _STEPS)
    out = _build(ctx, token_prefix, token_suffix, nl_t)
    return out, tokenized_prompts
